# baseline (speedup 1.0000x reference)
import jax
import jax.numpy as jnp
from jax import lax
from jax.experimental import pallas as pl
from jax.experimental.pallas import tpu as pltpu
from jax.experimental.pallas import tpu_sc as plsc

N = 32768
C = 64
NSC = 2
NSUB = 16
RPS = N // (NSC * NSUB)
SB = 256
NSB = RPS // SB
NBUF = 3
L = 16
CG = C // L
U = 16

_mesh = plsc.VectorSubcoreMesh(core_axis_name="c", subcore_axis_name="s")

_scratch_types = [
    pltpu.VMEM((SB, C), jnp.float32),
    pltpu.VMEM((SB, C), jnp.float32),
    pltpu.VMEM((SB, C), jnp.float32),
    pltpu.VMEM((C,), jnp.float32),
    pltpu.VMEM((C,), jnp.float32),
    pltpu.VMEM((3 * NSUB, C), jnp.float32),
    pltpu.SemaphoreType.DMA,
    pltpu.SemaphoreType.DMA,
    pltpu.SemaphoreType.DMA,
    pltpu.SemaphoreType.DMA,
    pltpu.SemaphoreType.DMA,
    pltpu.SemaphoreType.DMA,
]


def _cumsum_body(x_hbm, out_hbm, aux_hbm,
                 b0, b1, b2, tstage, tstage_rem, totv,
                 l0, l1, l2, s0, s1, s2):
    c = lax.axis_index("c")
    s = lax.axis_index("s")
    base = (c * NSUB + s) * RPS
    bufs = [b0, b1, b2]
    lsem = [l0, l1, l2]
    ssem = [s0, s1, s2]

    load_base = ([s * RPS + sb * SB for sb in range(NSB)]
                 + 2 * [base + sb * SB for sb in range(NSB)])

    handles = [None] * (3 * NSB)

    def start_load(i):
        b = i % NBUF
        handles[i] = pltpu.async_copy(
            x_hbm.at[pl.ds(load_base[i], SB)], bufs[b], lsem[b])

    def sum_block(buf, acc):
        def body(i, a):
            out = list(a)
            for u in range(U):
                r = i * U + u
                for g in range(CG):
                    out[g] = out[g] + buf[r, pl.ds(g * L, L)]
            return tuple(out)

        return lax.fori_loop(0, SB // U, body, acc)

    def scan_block(buf, cc):
        def body(i, cc):
            cc = list(cc)
            for u in range(U):
                r = i * U + u
                for g in range(CG):
                    cc[g] = cc[g] + buf[r, pl.ds(g * L, L)]
                    buf[r, pl.ds(g * L, L)] = cc[g]
            return tuple(cc)

        return lax.fori_loop(0, SB // U, body, cc)

    zero = tuple(jnp.zeros((L,), jnp.float32) for _ in range(CG))

    for i in range(NBUF):
        start_load(i)

    acc = zero
    for i in range(NSB):
        handles[i].wait()
        acc = sum_block(bufs[i % NBUF], acc)
        start_load(i + NBUF)
    for g in range(CG):
        tstage_rem[pl.ds(g * L, L)] = acc[g]
    pltpu.sync_copy(tstage_rem, aux_hbm.at[2 * NSUB + s])

    acc = zero
    for i in range(NSB, 2 * NSB):
        handles[i].wait()
        acc = sum_block(bufs[i % NBUF], acc)
        if i + NBUF < 3 * NSB - 1:
            start_load(i + NBUF)
    for g in range(CG):
        tstage[pl.ds(g * L, L)] = acc[g]
    pltpu.sync_copy(tstage, aux_hbm.at[c * NSUB + s])

    plsc.subcore_barrier()
    pltpu.sync_copy(aux_hbm, totv)

    mb = (c == 1).astype(jnp.float32)

    def tbody(j, acc):
        ms = (j < s).astype(jnp.float32)
        return tuple(
            acc[g] + totv[c * NSUB + j, pl.ds(g * L, L)] * ms
            + totv[2 * NSUB + j, pl.ds(g * L, L)] * mb
            for g in range(CG))

    carry = lax.fori_loop(0, NSUB, tbody,
                          tuple(jnp.full((L,), -1.0, jnp.float32)
                                for _ in range(CG)))

    store_handles = [None] * NSB
    for k in range(NSB):
        i = 2 * NSB + k
        b = i % NBUF
        if i == 3 * NSB - 1:
            store_handles[0].wait()
            start_load(i)
        handles[i].wait()
        carry = scan_block(bufs[b], carry)
        store_handles[k] = pltpu.async_copy(
            bufs[b], out_hbm.at[pl.ds(base + k * SB, SB)], ssem[b])
    for k in range(1, NSB):
        store_handles[k].wait()


_cumsum_sc = pl.kernel(
    _cumsum_body,
    mesh=_mesh,
    out_type=(
        jax.ShapeDtypeStruct((N, C), jnp.float32),
        jax.ShapeDtypeStruct((3 * NSUB, C), jnp.float32),
    ),
    scratch_types=_scratch_types,
)


def kernel(mask_i):
    out, _ = _cumsum_sc(mask_i)
    return out


# speedup vs baseline: 1.0187x; 1.0187x over previous
import jax
import jax.numpy as jnp
from jax import lax
from jax.experimental import pallas as pl
from jax.experimental.pallas import tpu as pltpu
from jax.experimental.pallas import tpu_sc as plsc

N = 32768
C = 64
NSC = 2
NSUB = 16
RPS = N // (NSC * NSUB)
SB = 256
NSB = RPS // SB
NBUF = 3
L = 16
CG = C // L
U = 8

_mesh = plsc.VectorSubcoreMesh(core_axis_name="c", subcore_axis_name="s")

_scratch_types = [
    pltpu.VMEM((SB, C), jnp.float32),
    pltpu.VMEM((SB, C), jnp.float32),
    pltpu.VMEM((SB, C), jnp.float32),
    pltpu.VMEM((C,), jnp.float32),
    pltpu.VMEM((C,), jnp.float32),
    pltpu.VMEM((3 * NSUB, C), jnp.float32),
    pltpu.SemaphoreType.DMA,
    pltpu.SemaphoreType.DMA,
    pltpu.SemaphoreType.DMA,
    pltpu.SemaphoreType.DMA,
    pltpu.SemaphoreType.DMA,
    pltpu.SemaphoreType.DMA,
]


def _cumsum_body(x_hbm, out_hbm, aux_hbm,
                 b0, b1, b2, tstage, tstage_rem, totv,
                 l0, l1, l2, s0, s1, s2):
    c = lax.axis_index("c")
    s = lax.axis_index("s")
    base = (c * NSUB + s) * RPS
    bufs = [b0, b1, b2]
    lsem = [l0, l1, l2]
    ssem = [s0, s1, s2]

    load_base = ([s * RPS + sb * SB for sb in range(NSB)]
                 + 2 * [base + sb * SB for sb in range(NSB)])

    handles = [None] * (3 * NSB)

    def start_load(i):
        b = i % NBUF
        handles[i] = pltpu.async_copy(
            x_hbm.at[pl.ds(load_base[i], SB)], bufs[b], lsem[b])

    def sum_block(buf, acc):
        def body(i, a):
            out = list(a)
            for u in range(U):
                r = i * U + u
                for g in range(CG):
                    out[g] = out[g] + buf[r, pl.ds(g * L, L)]
            return tuple(out)

        return lax.fori_loop(0, SB // U, body, acc)

    def scan_block(buf, cc):
        def body(i, cc):
            cc = list(cc)
            for u in range(U):
                r = i * U + u
                for g in range(CG):
                    cc[g] = cc[g] + buf[r, pl.ds(g * L, L)]
                    buf[r, pl.ds(g * L, L)] = cc[g]
            return tuple(cc)

        return lax.fori_loop(0, SB // U, body, cc)

    zero = tuple(jnp.zeros((L,), jnp.float32) for _ in range(CG))

    for i in range(NBUF):
        start_load(i)

    acc = zero
    for i in range(NSB):
        handles[i].wait()
        acc = sum_block(bufs[i % NBUF], acc)
        start_load(i + NBUF)
    for g in range(CG):
        tstage_rem[pl.ds(g * L, L)] = acc[g]
    pltpu.sync_copy(tstage_rem, aux_hbm.at[2 * NSUB + s])

    acc = zero
    for i in range(NSB, 2 * NSB):
        handles[i].wait()
        acc = sum_block(bufs[i % NBUF], acc)
        if i + NBUF < 3 * NSB - 1:
            start_load(i + NBUF)
    for g in range(CG):
        tstage[pl.ds(g * L, L)] = acc[g]
    pltpu.sync_copy(tstage, aux_hbm.at[c * NSUB + s])

    plsc.subcore_barrier()
    pltpu.sync_copy(aux_hbm, totv)

    mb = (c == 1).astype(jnp.float32)

    def tbody(j, acc):
        ms = (j < s).astype(jnp.float32)
        return tuple(
            acc[g] + totv[c * NSUB + j, pl.ds(g * L, L)] * ms
            + totv[2 * NSUB + j, pl.ds(g * L, L)] * mb
            for g in range(CG))

    carry = lax.fori_loop(0, NSUB, tbody,
                          tuple(jnp.full((L,), -1.0, jnp.float32)
                                for _ in range(CG)))

    store_handles = [None] * NSB
    for k in range(NSB):
        i = 2 * NSB + k
        b = i % NBUF
        if i == 3 * NSB - 1:
            store_handles[0].wait()
            start_load(i)
        handles[i].wait()
        carry = scan_block(bufs[b], carry)
        store_handles[k] = pltpu.async_copy(
            bufs[b], out_hbm.at[pl.ds(base + k * SB, SB)], ssem[b])
    for k in range(1, NSB):
        store_handles[k].wait()


_cumsum_sc = pl.kernel(
    _cumsum_body,
    mesh=_mesh,
    compiler_params=pltpu.CompilerParams(
        disable_bounds_checks=True,
        disable_semaphore_checks=True,
    ),
    out_type=(
        jax.ShapeDtypeStruct((N, C), jnp.float32),
        jax.ShapeDtypeStruct((3 * NSUB, C), jnp.float32),
    ),
    scratch_types=_scratch_types,
)


def kernel(mask_i):
    out, _ = _cumsum_sc(mask_i)
    return out
